# baseline (speedup 1.0000x reference)
import functools

import jax
import jax.numpy as jnp
from jax import lax
from jax.experimental import pallas as pl
from jax.experimental.pallas import tpu as pltpu
from jax.experimental.pallas import tpu_sc as plsc

F32 = jnp.float32
I32 = jnp.int32
HI = lax.Precision.HIGHEST

N1 = 34
RPAD = 1216
BS = 1024
SL = 200
NPOS = BS * SL
NW = 32

CH = 400
NCH = NPOS // (NW * CH)

BB = 32
NBLK = BS // BB


def _relfull_body(tab_ref, out_ref):
    tab = tab_ref[...]
    i = lax.broadcasted_iota(I32, (RPAD, RPAD), 0)
    j = lax.broadcasted_iota(I32, (RPAD, RPAD), 1)
    tr = jnp.maximum(i - N1, 0)
    r1 = tr // N1
    r2 = tr - r1 * N1
    s1 = (j == r1).astype(F32)
    s2 = (j == r2).astype(F32)
    two = (jnp.dot(s1, tab, preferred_element_type=F32, precision=HI)
           * jnp.dot(s2, tab, preferred_element_type=F32, precision=HI))
    row = lax.broadcasted_iota(I32, (RPAD, 1), 0)
    out_ref[...] = jnp.where(row < N1, tab, two)


def _build_rel_full(rel_table):
    tab = jnp.pad(rel_table, ((0, RPAD - rel_table.shape[0]), (0, 0)))
    return pl.pallas_call(
        _relfull_body,
        out_shape=jax.ShapeDtypeStruct((RPAD, 64), F32),
    )(tab)


def _sc_gather(ctab, q_idx, a_idx, rtab, rel_idx):
    mesh = plsc.VectorSubcoreMesh(core_axis_name="c", subcore_axis_name="s")

    @functools.partial(
        pl.kernel,
        mesh=mesh,
        out_type=(
            jax.ShapeDtypeStruct((NW * NCH, CH, 128), F32),
            jax.ShapeDtypeStruct((NW * NCH, CH, 128), F32),
        ),
        scratch_types=[
            pltpu.VMEM((4, CH // 4), I32),
            pltpu.VMEM((CH, 64), F32),
            pltpu.SemaphoreType.DMA,
        ],
        compiler_params=pltpu.CompilerParams(use_tc_tiling_on_sc=False),
    )
    def k(ctab_h, q_idx_h, a_idx_h, rtab_h, rel_idx_h,
          qa_out, r_out, i_v, rows_v, sem):
        wid = lax.axis_index("s") * 2 + lax.axis_index("c")

        jobs = ((q_idx_h, qa_out, ctab_h, 0),
                (a_idx_h, qa_out, ctab_h, 64),
                (rel_idx_h, r_out, rtab_h, 0))
        for idx_h, out_h, tab_h, col in jobs:
            def body(i, carry, idx_h=idx_h, out_h=out_h, tab_h=tab_h,
                     col=col):
                ci = wid * NCH + i
                pltpu.sync_copy(idx_h.at[ci], i_v)
                cps = [
                    pltpu.async_copy(
                        tab_h.at[i_v.at[j]],
                        rows_v.at[pl.ds(j * (CH // 4), CH // 4)], sem)
                    for j in range(4)
                ]
                for c in cps:
                    c.wait()
                pltpu.sync_copy(rows_v, out_h.at[ci, :, pl.ds(col, 64)])
                return carry

            lax.fori_loop(0, NCH, body, 0)

    return k(ctab, q_idx, a_idx, rtab, rel_idx)


def _mlp_body(qa_ref, rel_ref, nt_ref, path_ref,
              w0qa_ref, w0r_ref, b0_ref, g0_ref, be0_ref,
              w1_ref, b1_ref,
              o0p_ref, o0q_ref, ob0_ref, og0_ref, obe0_ref, o1_ref, ob1_ref,
              out_ref):
    npos = BB * SL
    bf = jnp.bfloat16
    h = (jnp.dot(qa_ref[...].astype(bf), w0qa_ref[...].astype(bf),
                 preferred_element_type=F32)
         + jnp.dot(rel_ref[:, :64].astype(bf), w0r_ref[...].astype(bf),
                   preferred_element_type=F32)
         + b0_ref[...])
    mu = jnp.mean(h, axis=1, keepdims=True)
    var = jnp.mean((h - mu) * (h - mu), axis=1, keepdims=True)
    h = (h - mu) / jnp.sqrt(var + 1e-5) * g0_ref[...] + be0_ref[...]
    h = 0.5 * h * (1.0 + lax.erf(h * 0.7071067811865476))
    qars = (jnp.dot(h.astype(bf), w1_ref[...].astype(bf),
                    preferred_element_type=F32)
            + b1_ref[...])

    nt = jnp.maximum(nt_ref[0], 1)
    s3 = lax.broadcasted_iota(I32, (BB, SL, 128), 1)
    q3 = jnp.where(s3 < nt[:, :, None], qars.reshape(BB, SL, 128), 0.0)
    pooled = q3.sum(axis=1) / nt.astype(F32)

    h2 = (jnp.dot(path_ref[...], o0p_ref[...], preferred_element_type=F32)
          + jnp.dot(pooled, o0q_ref[...], preferred_element_type=F32)
          + ob0_ref[...])
    mu2 = jnp.mean(h2, axis=1, keepdims=True)
    var2 = jnp.mean((h2 - mu2) * (h2 - mu2), axis=1, keepdims=True)
    h2 = (h2 - mu2) / jnp.sqrt(var2 + 1e-5) * og0_ref[...] + obe0_ref[...]
    h2 = 0.5 * h2 * (1.0 + lax.erf(h2 * 0.7071067811865476))
    out_ref[...] = (jnp.dot(h2, o1_ref[...], preferred_element_type=F32)
                    + ob1_ref[...])


def _fused_mlp(qa2, rel2, nt4, path, w0qa, w0r, b0, g0, be0, w1, b1,
               o0p, o0q, ob0, og0, obe0, o1, ob1):
    npos = BB * SL
    full = lambda shape: pl.BlockSpec(shape, lambda i: (0,) * len(shape))
    return pl.pallas_call(
        _mlp_body,
        grid=(NBLK,),
        in_specs=[
            pl.BlockSpec((npos, 128), lambda i: (i, 0)),
            pl.BlockSpec((npos, 128), lambda i: (i, 0)),
            pl.BlockSpec((1, BB, 1), lambda i: (i, 0, 0)),
            pl.BlockSpec((BB, 768), lambda i: (i, 0)),
            full((128, 256)), full((64, 256)),
            full((1, 256)), full((1, 256)), full((1, 256)),
            full((256, 128)), full((1, 128)),
            full((768, 256)), full((128, 256)), full((1, 256)),
            full((1, 256)), full((1, 256)), full((256, 128)), full((1, 128)),
        ],
        out_specs=pl.BlockSpec((BB, 128), lambda i: (i, 0)),
        out_shape=jax.ShapeDtypeStruct((BS, 128), F32),
        compiler_params=pltpu.CompilerParams(
            dimension_semantics=("arbitrary",)),
    )(qa2, rel2, nt4, path, w0qa, w0r, b0, g0, be0, w1, b1,
      o0p, o0q, ob0, og0, obe0, o1, ob1)


def kernel(path_embedding, sent_vecs, qa_ids, rel_ids, num_tuples,
           concept_table, rel_table,
           mlp_W0, mlp_b0, mlp_g0, mlp_be0, mlp_W1, mlp_b1,
           out_W0, out_b0, out_g0, out_be0, out_W1, out_b1):
    qa = qa_ids.astype(I32)
    q_idx = qa[:, :, 0].reshape(NW * NCH, 4, CH // 4)
    a_idx = qa[:, :, 1].reshape(NW * NCH, 4, CH // 4)
    rel_idx = rel_ids.astype(I32).reshape(NW * NCH, 4, CH // 4)
    nt4 = num_tuples.astype(I32).reshape(NBLK, BB, 1)

    rel_full = _build_rel_full(rel_table)
    qa_rows, rel_rows = _sc_gather(concept_table, q_idx, a_idx,
                                   rel_full, rel_idx)
    qa2 = qa_rows.reshape(NPOS, 128)
    rel2 = rel_rows.reshape(NPOS, 128)

    r2 = lambda v: v.reshape(1, -1)
    return _fused_mlp(
        qa2, rel2, nt4, path_embedding,
        mlp_W0[:128], mlp_W0[128:],
        r2(mlp_b0), r2(mlp_g0), r2(mlp_be0),
        mlp_W1, r2(mlp_b1),
        out_W0[:768], out_W0[768:], r2(out_b0), r2(out_g0), r2(out_be0),
        out_W1, r2(out_b1))


# speedup vs baseline: 1.0192x; 1.0192x over previous
import functools

import jax
import jax.numpy as jnp
from jax import lax
from jax.experimental import pallas as pl
from jax.experimental.pallas import tpu as pltpu
from jax.experimental.pallas import tpu_sc as plsc

F32 = jnp.float32
I32 = jnp.int32
HI = lax.Precision.HIGHEST

N1 = 34
RPAD = 1216
BS = 1024
SL = 200
NPOS = BS * SL
NW = 32

CH = 400
NCH = NPOS // (NW * CH)

BB = 32
NBLK = BS // BB


def _relfull_body(tab_ref, out_ref):
    tab = tab_ref[...]
    i = lax.broadcasted_iota(I32, (RPAD, RPAD), 0)
    j = lax.broadcasted_iota(I32, (RPAD, RPAD), 1)
    tr = jnp.maximum(i - N1, 0)
    r1 = tr // N1
    r2 = tr - r1 * N1
    s1 = (j == r1).astype(F32)
    s2 = (j == r2).astype(F32)
    two = (jnp.dot(s1, tab, preferred_element_type=F32, precision=HI)
           * jnp.dot(s2, tab, preferred_element_type=F32, precision=HI))
    row = lax.broadcasted_iota(I32, (RPAD, 1), 0)
    out_ref[...] = jnp.where(row < N1, tab, two)


def _build_rel_full(rel_table):
    tab = jnp.pad(rel_table, ((0, RPAD - rel_table.shape[0]), (0, 0)))
    return pl.pallas_call(
        _relfull_body,
        out_shape=jax.ShapeDtypeStruct((RPAD, 64), F32),
    )(tab)


def _sc_gather(ctab, q_idx, a_idx, rtab, rel_idx):
    mesh = plsc.VectorSubcoreMesh(core_axis_name="c", subcore_axis_name="s")

    @functools.partial(
        pl.kernel,
        mesh=mesh,
        out_type=(
            jax.ShapeDtypeStruct((NW * NCH, CH, 128), F32),
            jax.ShapeDtypeStruct((NW * NCH, CH, 128), F32),
        ),
        scratch_types=[
            pltpu.VMEM((4, CH // 4), I32),
            pltpu.VMEM((CH, 64), F32),
            pltpu.SemaphoreType.DMA,
        ],
        compiler_params=pltpu.CompilerParams(use_tc_tiling_on_sc=False),
    )
    def k(ctab_h, q_idx_h, a_idx_h, rtab_h, rel_idx_h,
          qa_out, r_out, i_v, rows_v, sem):
        wid = lax.axis_index("s") * 2 + lax.axis_index("c")

        jobs = ((q_idx_h, qa_out, ctab_h, 0),
                (a_idx_h, qa_out, ctab_h, 64),
                (rel_idx_h, r_out, rtab_h, 0))
        for idx_h, out_h, tab_h, col in jobs:
            def body(i, carry, idx_h=idx_h, out_h=out_h, tab_h=tab_h,
                     col=col):
                ci = wid * NCH + i
                pltpu.sync_copy(idx_h.at[ci], i_v)
                cps = [
                    pltpu.async_copy(
                        tab_h.at[i_v.at[j]],
                        rows_v.at[pl.ds(j * (CH // 4), CH // 4)], sem)
                    for j in range(4)
                ]
                for c in cps:
                    c.wait()
                pltpu.sync_copy(rows_v, out_h.at[ci, :, pl.ds(col, 64)])
                return carry

            lax.fori_loop(0, NCH, body, 0)

    return k(ctab, q_idx, a_idx, rtab, rel_idx)


def _mlp_body(qa_ref, rel_ref, nt_ref, path_ref,
              w0qa_ref, w0r_ref, b0_ref, g0_ref, be0_ref,
              w1_ref, b1_ref,
              o0p_ref, o0q_ref, ob0_ref, og0_ref, obe0_ref, o1_ref, ob1_ref,
              out_ref):
    npos = BB * SL
    h = (jnp.dot(qa_ref[...], w0qa_ref[...], preferred_element_type=F32)
         + jnp.dot(rel_ref[:, :64], w0r_ref[...], preferred_element_type=F32)
         + b0_ref[...])
    mu = jnp.mean(h, axis=1, keepdims=True)
    var = jnp.mean((h - mu) * (h - mu), axis=1, keepdims=True)
    h = (h - mu) / jnp.sqrt(var + 1e-5) * g0_ref[...] + be0_ref[...]
    h = 0.5 * h * (1.0 + lax.erf(h * 0.7071067811865476))
    qars = (jnp.dot(h, w1_ref[...], preferred_element_type=F32)
            + b1_ref[...])

    nt = jnp.maximum(nt_ref[0], 1)
    s3 = lax.broadcasted_iota(I32, (BB, SL, 128), 1)
    q3 = jnp.where(s3 < nt[:, :, None], qars.reshape(BB, SL, 128), 0.0)
    pooled = q3.sum(axis=1) / nt.astype(F32)

    h2 = (jnp.dot(path_ref[...], o0p_ref[...], preferred_element_type=F32)
          + jnp.dot(pooled, o0q_ref[...], preferred_element_type=F32)
          + ob0_ref[...])
    mu2 = jnp.mean(h2, axis=1, keepdims=True)
    var2 = jnp.mean((h2 - mu2) * (h2 - mu2), axis=1, keepdims=True)
    h2 = (h2 - mu2) / jnp.sqrt(var2 + 1e-5) * og0_ref[...] + obe0_ref[...]
    h2 = 0.5 * h2 * (1.0 + lax.erf(h2 * 0.7071067811865476))
    out_ref[...] = (jnp.dot(h2, o1_ref[...], preferred_element_type=F32)
                    + ob1_ref[...])


def _fused_mlp(qa2, rel2, nt4, path, w0qa, w0r, b0, g0, be0, w1, b1,
               o0p, o0q, ob0, og0, obe0, o1, ob1):
    npos = BB * SL
    full = lambda shape: pl.BlockSpec(shape, lambda i: (0,) * len(shape))
    return pl.pallas_call(
        _mlp_body,
        grid=(NBLK,),
        in_specs=[
            pl.BlockSpec((npos, 128), lambda i: (i, 0)),
            pl.BlockSpec((npos, 128), lambda i: (i, 0)),
            pl.BlockSpec((1, BB, 1), lambda i: (i, 0, 0)),
            pl.BlockSpec((BB, 768), lambda i: (i, 0)),
            full((128, 256)), full((64, 256)),
            full((1, 256)), full((1, 256)), full((1, 256)),
            full((256, 128)), full((1, 128)),
            full((768, 256)), full((128, 256)), full((1, 256)),
            full((1, 256)), full((1, 256)), full((256, 128)), full((1, 128)),
        ],
        out_specs=pl.BlockSpec((BB, 128), lambda i: (i, 0)),
        out_shape=jax.ShapeDtypeStruct((BS, 128), F32),
        compiler_params=pltpu.CompilerParams(
            dimension_semantics=("arbitrary",)),
    )(qa2, rel2, nt4, path, w0qa, w0r, b0, g0, be0, w1, b1,
      o0p, o0q, ob0, og0, obe0, o1, ob1)


def kernel(path_embedding, sent_vecs, qa_ids, rel_ids, num_tuples,
           concept_table, rel_table,
           mlp_W0, mlp_b0, mlp_g0, mlp_be0, mlp_W1, mlp_b1,
           out_W0, out_b0, out_g0, out_be0, out_W1, out_b1):
    qa = qa_ids.astype(I32)
    q_idx = qa[:, :, 0].reshape(NW * NCH, 4, CH // 4)
    a_idx = qa[:, :, 1].reshape(NW * NCH, 4, CH // 4)
    rel_idx = rel_ids.astype(I32).reshape(NW * NCH, 4, CH // 4)
    nt4 = num_tuples.astype(I32).reshape(NBLK, BB, 1)

    rel_full = _build_rel_full(rel_table)
    qa_rows, rel_rows = _sc_gather(concept_table, q_idx, a_idx,
                                   rel_full, rel_idx)
    qa2 = qa_rows.reshape(NPOS, 128)
    rel2 = rel_rows.reshape(NPOS, 128)

    r2 = lambda v: v.reshape(1, -1)
    return _fused_mlp(
        qa2, rel2, nt4, path_embedding,
        mlp_W0[:128], mlp_W0[128:],
        r2(mlp_b0), r2(mlp_g0), r2(mlp_be0),
        mlp_W1, r2(mlp_b1),
        out_W0[:768], out_W0[768:], r2(out_b0), r2(out_g0), r2(out_be0),
        out_W1, r2(out_b1))


# speedup vs baseline: 1.0304x; 1.0110x over previous
import functools

import jax
import jax.numpy as jnp
from jax import lax
from jax.experimental import pallas as pl
from jax.experimental.pallas import tpu as pltpu
from jax.experimental.pallas import tpu_sc as plsc

F32 = jnp.float32
I32 = jnp.int32
HI = lax.Precision.HIGHEST

N1 = 34
RPAD = 1216
BS = 1024
SL = 200
NPOS = BS * SL
NW = 32

CH = 400
NCH = NPOS // (NW * CH)

BB = 32
NBLK = BS // BB


def _relfull_body(tab_ref, out_ref):
    tab = tab_ref[...]
    i = lax.broadcasted_iota(I32, (RPAD, RPAD), 0)
    j = lax.broadcasted_iota(I32, (RPAD, RPAD), 1)
    tr = jnp.maximum(i - N1, 0)
    r1 = tr // N1
    r2 = tr - r1 * N1
    s1 = (j == r1).astype(F32)
    s2 = (j == r2).astype(F32)
    two = (jnp.dot(s1, tab, preferred_element_type=F32, precision=HI)
           * jnp.dot(s2, tab, preferred_element_type=F32, precision=HI))
    row = lax.broadcasted_iota(I32, (RPAD, 1), 0)
    out_ref[...] = jnp.where(row < N1, tab, two)


def _build_rel_full(rel_table):
    tab = jnp.pad(rel_table, ((0, RPAD - rel_table.shape[0]), (0, 0)))
    return pl.pallas_call(
        _relfull_body,
        out_shape=jax.ShapeDtypeStruct((RPAD, 64), F32),
    )(tab)


def _sc_gather(ctab, q_idx, a_idx, rtab, rel_idx):
    mesh = plsc.VectorSubcoreMesh(core_axis_name="c", subcore_axis_name="s")

    @functools.partial(
        pl.kernel,
        mesh=mesh,
        out_type=(
            jax.ShapeDtypeStruct((NW * NCH, CH, 128), F32),
            jax.ShapeDtypeStruct((NW * NCH, CH, 128), F32),
        ),
        scratch_types=[
            pltpu.VMEM((4, CH // 4), I32),
            pltpu.VMEM((4, CH // 4), I32),
            pltpu.VMEM((CH, 64), F32),
            pltpu.VMEM((CH, 64), F32),
            pltpu.SemaphoreType.DMA,
            pltpu.SemaphoreType.DMA,
        ],
        compiler_params=pltpu.CompilerParams(use_tc_tiling_on_sc=False),
    )
    def k(ctab_h, q_idx_h, a_idx_h, rtab_h, rel_idx_h,
          qa_out, r_out, i_v0, i_v1, rows_v0, rows_v1, sem, sem_w):
        wid = lax.axis_index("s") * 2 + lax.axis_index("c")

        jobs = ((q_idx_h, qa_out, ctab_h, 0),
                (a_idx_h, qa_out, ctab_h, 64),
                (rel_idx_h, r_out, rtab_h, 0))
        for idx_h, out_h, tab_h, col in jobs:
            def body(i, carry, idx_h=idx_h, out_h=out_h, tab_h=tab_h,
                     col=col):
                ci0 = wid * NCH + 2 * i
                pltpu.sync_copy(idx_h.at[ci0], i_v0)
                cps0 = [
                    pltpu.async_copy(
                        tab_h.at[i_v0.at[j]],
                        rows_v0.at[pl.ds(j * (CH // 4), CH // 4)], sem)
                    for j in range(4)
                ]
                pltpu.sync_copy(idx_h.at[ci0 + 1], i_v1)
                for c in cps0:
                    c.wait()
                w0 = pltpu.async_copy(
                    rows_v0, out_h.at[ci0, :, pl.ds(col, 64)], sem_w)
                cps1 = [
                    pltpu.async_copy(
                        tab_h.at[i_v1.at[j]],
                        rows_v1.at[pl.ds(j * (CH // 4), CH // 4)], sem)
                    for j in range(4)
                ]
                for c in cps1:
                    c.wait()
                w1 = pltpu.async_copy(
                    rows_v1, out_h.at[ci0 + 1, :, pl.ds(col, 64)], sem_w)
                w0.wait()
                w1.wait()
                return carry

            lax.fori_loop(0, NCH // 2, body, 0)

    return k(ctab, q_idx, a_idx, rtab, rel_idx)


def _mlp_body(qa_ref, rel_ref, nt_ref, path_ref,
              w0qa_ref, w0r_ref, b0_ref, g0_ref, be0_ref,
              w1_ref, b1_ref,
              o0p_ref, o0q_ref, ob0_ref, og0_ref, obe0_ref, o1_ref, ob1_ref,
              out_ref):
    npos = BB * SL
    h = (jnp.dot(qa_ref[...], w0qa_ref[...], preferred_element_type=F32)
         + jnp.dot(rel_ref[:, :64], w0r_ref[...], preferred_element_type=F32)
         + b0_ref[...])
    mu = jnp.mean(h, axis=1, keepdims=True)
    var = jnp.mean((h - mu) * (h - mu), axis=1, keepdims=True)
    h = (h - mu) / jnp.sqrt(var + 1e-5) * g0_ref[...] + be0_ref[...]
    h = 0.5 * h * (1.0 + lax.erf(h * 0.7071067811865476))
    qars = (jnp.dot(h, w1_ref[...], preferred_element_type=F32)
            + b1_ref[...])

    nt = jnp.maximum(nt_ref[0], 1)
    s3 = lax.broadcasted_iota(I32, (BB, SL, 128), 1)
    q3 = jnp.where(s3 < nt[:, :, None], qars.reshape(BB, SL, 128), 0.0)
    pooled = q3.sum(axis=1) / nt.astype(F32)

    h2 = (jnp.dot(path_ref[...], o0p_ref[...], preferred_element_type=F32)
          + jnp.dot(pooled, o0q_ref[...], preferred_element_type=F32)
          + ob0_ref[...])
    mu2 = jnp.mean(h2, axis=1, keepdims=True)
    var2 = jnp.mean((h2 - mu2) * (h2 - mu2), axis=1, keepdims=True)
    h2 = (h2 - mu2) / jnp.sqrt(var2 + 1e-5) * og0_ref[...] + obe0_ref[...]
    h2 = 0.5 * h2 * (1.0 + lax.erf(h2 * 0.7071067811865476))
    out_ref[...] = (jnp.dot(h2, o1_ref[...], preferred_element_type=F32)
                    + ob1_ref[...])


def _fused_mlp(qa2, rel2, nt4, path, w0qa, w0r, b0, g0, be0, w1, b1,
               o0p, o0q, ob0, og0, obe0, o1, ob1):
    npos = BB * SL
    full = lambda shape: pl.BlockSpec(shape, lambda i: (0,) * len(shape))
    return pl.pallas_call(
        _mlp_body,
        grid=(NBLK,),
        in_specs=[
            pl.BlockSpec((npos, 128), lambda i: (i, 0)),
            pl.BlockSpec((npos, 128), lambda i: (i, 0)),
            pl.BlockSpec((1, BB, 1), lambda i: (i, 0, 0)),
            pl.BlockSpec((BB, 768), lambda i: (i, 0)),
            full((128, 256)), full((64, 256)),
            full((1, 256)), full((1, 256)), full((1, 256)),
            full((256, 128)), full((1, 128)),
            full((768, 256)), full((128, 256)), full((1, 256)),
            full((1, 256)), full((1, 256)), full((256, 128)), full((1, 128)),
        ],
        out_specs=pl.BlockSpec((BB, 128), lambda i: (i, 0)),
        out_shape=jax.ShapeDtypeStruct((BS, 128), F32),
        compiler_params=pltpu.CompilerParams(
            dimension_semantics=("arbitrary",)),
    )(qa2, rel2, nt4, path, w0qa, w0r, b0, g0, be0, w1, b1,
      o0p, o0q, ob0, og0, obe0, o1, ob1)


def kernel(path_embedding, sent_vecs, qa_ids, rel_ids, num_tuples,
           concept_table, rel_table,
           mlp_W0, mlp_b0, mlp_g0, mlp_be0, mlp_W1, mlp_b1,
           out_W0, out_b0, out_g0, out_be0, out_W1, out_b1):
    qa = qa_ids.astype(I32)
    q_idx = qa[:, :, 0].reshape(NW * NCH, 4, CH // 4)
    a_idx = qa[:, :, 1].reshape(NW * NCH, 4, CH // 4)
    rel_idx = rel_ids.astype(I32).reshape(NW * NCH, 4, CH // 4)
    nt4 = num_tuples.astype(I32).reshape(NBLK, BB, 1)

    rel_full = _build_rel_full(rel_table)
    qa_rows, rel_rows = _sc_gather(concept_table, q_idx, a_idx,
                                   rel_full, rel_idx)
    qa2 = qa_rows.reshape(NPOS, 128)
    rel2 = rel_rows.reshape(NPOS, 128)

    r2 = lambda v: v.reshape(1, -1)
    return _fused_mlp(
        qa2, rel2, nt4, path_embedding,
        mlp_W0[:128], mlp_W0[128:],
        r2(mlp_b0), r2(mlp_g0), r2(mlp_be0),
        mlp_W1, r2(mlp_b1),
        out_W0[:768], out_W0[768:], r2(out_b0), r2(out_g0), r2(out_be0),
        out_W1, r2(out_b1))
